# baseline (speedup 1.0000x reference)
import functools

import jax
import jax.numpy as jnp
from jax import lax
from jax.experimental import pallas as pl
from jax.experimental.pallas import tpu as pltpu
from jax.experimental.pallas import tpu_sc as plsc

V = 1000
SL = 848
VR = V - SL
N_TOK = 204800
NC, NS, L = 2, 16, 16
NW = NC * NS
NB = N_TOK // NW
CH = 16
NCHUNK = NB // CH
CB = V // 8
CBL = SL // 8
CBR = VR // 8
RB = N_TOK // 128


def _lse_body(tab_ref, lse_ref):
    x = tab_ref[...]
    m = jnp.max(x, axis=1, keepdims=True)
    s = jnp.sum(jnp.exp(x - m), axis=1, keepdims=True)
    lse_ref[...] = m + jnp.log(s)


def _row_logsumexp(table):
    return pl.pallas_call(
        _lse_body,
        out_shape=jax.ShapeDtypeStruct((V, 1), jnp.float32),
    )(table).reshape(V)


def _sc_body(tabl_hbm, tabr_hbm, ctx_hbm, tgt_hbm, lse_hbm,
             out_hbm, part_hbm,
             ctx_v, tgt_v, lse_v, rows0, rows1,
             rowsr0, rowsr1, stg0, stg1, acc, tab_sh,
             in0, in1, out0, out1):
    sid = lax.axis_index("s")
    wid = sid * NC + lax.axis_index("c")
    base = wid * NB
    rb_base = base // 128

    for k in range(8):
        chunk = sid + 16 * k

        @pl.when(chunk < 125)
        def _():
            pltpu.sync_copy(tabl_hbm.at[pl.ds(chunk * 8, 8)],
                            rows0.at[pl.ds(0, 8)])
            pltpu.sync_copy(rows0.at[pl.ds(0, 8)],
                            tab_sh.at[pl.ds(chunk * 8, 8)])

    pltpu.sync_copy(ctx_hbm.at[pl.ds(base, NB)], ctx_v)
    pltpu.sync_copy(tgt_hbm.at[pl.ds(base, NB)], tgt_v)
    pltpu.sync_copy(lse_hbm, lse_v)
    acc[...] = jnp.zeros((L,), jnp.float32)
    plsc.subcore_barrier()

    rows = (rows0, rows1)
    rowsr = (rowsr0, rowsr1)
    stg = (stg0, stg1)
    sin = (in0, in1)
    sout = (out0, out1)
    lane = lax.iota(jnp.int32, L)

    def start_gather(g, p):
        idx = ctx_v.at[pl.ds(g * CH, CH)]
        pltpu.make_async_copy(tabl_hbm.at[idx], rows[p], sin[p]).start()
        pltpu.make_async_copy(tabr_hbm.at[idx], rowsr[p], sin[p]).start()

    def wait_gather(p):
        idx = ctx_v.at[pl.ds(0, CH)]
        pltpu.make_async_copy(tabl_hbm.at[idx], rows[p], sin[p]).wait()
        pltpu.make_async_copy(tabr_hbm.at[idx], rowsr[p], sin[p]).wait()

    def start_out(g, p):
        tok = base + g * CH
        rb = tok // 128
        r_off = tok % 128
        pltpu.make_async_copy(
            stg[p], out_hbm.at[:, rb, :, pl.ds(r_off, CH)], sout[p]
        ).start()

    def wait_out(p):
        pltpu.make_async_copy(
            stg[p], out_hbm.at[:, rb_base, :, pl.ds(0, CH)], sout[p]
        ).wait()

    def transpose_loss(g, p):
        off = g * CH
        ctx16 = ctx_v[pl.ds(off, L)]
        tgt16 = tgt_v[pl.ds(off, L)]
        t_l = jnp.minimum(tgt16, SL - 1)
        t_r = jnp.maximum(tgt16 - SL, 0)
        vals_l = plsc.load_gather(rows[p], [lane, t_l])
        vals_r = plsc.load_gather(rowsr[p], [lane, t_r])
        vals = jnp.where(tgt16 < SL, vals_l, vals_r)
        lsec = plsc.load_gather(lse_v, [ctx16])
        acc[...] = acc[...] + (lsec - vals)

        @plsc.parallel_loop(0, CBL, unroll=4)
        def cb_body(cb):
            c0 = cb * 8
            for c in range(8):
                col = jnp.full((L,), c0 + c, jnp.int32)
                stg[p][cb, c, :] = plsc.load_gather(rows[p], [lane, col])

        @plsc.parallel_loop(0, CBR, unroll=4)
        def cbr_body(cb):
            c0 = cb * 8
            for c in range(8):
                col = jnp.full((L,), c0 + c, jnp.int32)
                stg[p][CBL + cb, c, :] = plsc.load_gather(
                    rowsr[p], [lane, col])

    start_gather(0, 0)
    start_gather(1, 1)
    for g in (0, 1):
        p = g % 2
        wait_gather(p)
        transpose_loss(g, p)
        start_out(g, p)
        start_gather(g + 2, p)

    def pair_body(g2, _):
        for k in range(2):
            g = 2 * g2 + k
            p = k
            wait_gather(p)
            wait_out(p)
            transpose_loss(g, p)
            start_out(g, p)
            start_gather(g + 2, p)
        return 0

    lax.fori_loop(1, NCHUNK // 2 - 1, pair_body, 0)

    for g in (NCHUNK - 2, NCHUNK - 1):
        p = g % 2
        wait_gather(p)
        wait_out(p)
        transpose_loss(g, p)
        start_out(g, p)

    wait_out(0)
    wait_out(1)
    pltpu.sync_copy(acc, part_hbm.at[pl.ds(wid * L, L)])


@functools.partial(jax.jit, static_argnums=())
def kernel(context, targets, table):
    ctx_flat = context.reshape(N_TOK)
    tgt_flat = targets.reshape(N_TOK)
    tabl = table[:, :SL]
    tabr = table[:, SL:]
    lse = _row_logsumexp(table)

    mesh = plsc.VectorSubcoreMesh(core_axis_name="c", subcore_axis_name="s")
    sc = pl.kernel(
        _sc_body,
        out_type=(
            jax.ShapeDtypeStruct((CB, RB, 8, 128), jnp.float32),
            jax.ShapeDtypeStruct((NW * L,), jnp.float32),
        ),
        mesh=mesh,
        compiler_params=pltpu.CompilerParams(
            needs_layout_passes=False, use_tc_tiling_on_sc=False),
        scratch_types=[
            pltpu.VMEM((NB,), jnp.int32),
            pltpu.VMEM((NB,), jnp.int32),
            pltpu.VMEM((V,), jnp.float32),
            pltpu.VMEM((CH, SL), jnp.float32),
            pltpu.VMEM((CH, SL), jnp.float32),
            pltpu.VMEM((CH, VR), jnp.float32),
            pltpu.VMEM((CH, VR), jnp.float32),
            pltpu.VMEM((CB, 8, CH), jnp.float32),
            pltpu.VMEM((CB, 8, CH), jnp.float32),
            pltpu.VMEM((L,), jnp.float32),
            pltpu.VMEM_SHARED((V, SL), jnp.float32),
            pltpu.SemaphoreType.DMA,
            pltpu.SemaphoreType.DMA,
            pltpu.SemaphoreType.DMA,
            pltpu.SemaphoreType.DMA,
        ],
    )
    out4, partials = sc(tabl, tabr, ctx_flat, tgt_flat, lse)
    logits2 = out4.transpose(1, 3, 0, 2).reshape(N_TOK, V)
    loss = jnp.sum(partials) / jnp.float32(N_TOK)
    return (logits2, loss)


# speedup vs baseline: 1.6622x; 1.6622x over previous
import functools

import jax
import jax.numpy as jnp
from jax import lax
from jax.experimental import pallas as pl
from jax.experimental.pallas import tpu as pltpu
from jax.experimental.pallas import tpu_sc as plsc

V = 1000
N_TOK = 204800
NC, NS, L = 2, 16, 16
NW = NC * NS
NB = N_TOK // NW
CH = 16
NCHUNK = NB // CH
CB = V // 8
RB = N_TOK // 128


def _lse_body(tab_ref, lse_ref):
    x = tab_ref[...]
    m = jnp.max(x, axis=1, keepdims=True)
    s = jnp.sum(jnp.exp(x - m), axis=1, keepdims=True)
    lse_ref[...] = m + jnp.log(s)


def _row_logsumexp(table):
    return pl.pallas_call(
        _lse_body,
        out_shape=jax.ShapeDtypeStruct((V, 1), jnp.float32),
    )(table).reshape(V)


def _sc_body(table_hbm, ctx_hbm, tgt_hbm, lse_hbm,
             out_hbm, part_hbm,
             ctx_v, tgt_v, lse_v, rows0, rows1,
             stg0, stg1, acc,
             in0, in1, out0, out1):
    wid = lax.axis_index("s") * NC + lax.axis_index("c")
    base = wid * NB
    rb_base = base // 128

    pltpu.sync_copy(ctx_hbm.at[pl.ds(base, NB)], ctx_v)
    pltpu.sync_copy(tgt_hbm.at[pl.ds(base, NB)], tgt_v)
    pltpu.sync_copy(lse_hbm, lse_v)
    acc[...] = jnp.zeros((L,), jnp.float32)

    rows = (rows0, rows1)
    stg = (stg0, stg1)
    sin = (in0, in1)
    sout = (out0, out1)
    lane = lax.iota(jnp.int32, L)

    def start_gather(g, p):
        pltpu.make_async_copy(
            table_hbm.at[ctx_v.at[pl.ds(g * CH, CH)]], rows[p], sin[p]
        ).start()

    def wait_gather(p):
        pltpu.make_async_copy(
            table_hbm.at[ctx_v.at[pl.ds(0, CH)]], rows[p], sin[p]
        ).wait()

    def start_out(g_first, ps):
        tok = base + g_first * CH
        rb = tok // 128
        r_off = tok % 128
        pltpu.make_async_copy(
            stg[ps], out_hbm.at[:, rb, :, pl.ds(r_off, 2 * CH)], sout[ps]
        ).start()

    def wait_out(ps):
        pltpu.make_async_copy(
            stg[ps], out_hbm.at[:, rb_base, :, pl.ds(0, 2 * CH)], sout[ps]
        ).wait()

    def transpose_loss(g, p, ps, h):
        off = g * CH
        ctx16 = ctx_v[pl.ds(off, L)]
        tgt16 = tgt_v[pl.ds(off, L)]
        vals = plsc.load_gather(rows[p], [lane, tgt16])
        lsec = plsc.load_gather(lse_v, [ctx16])
        acc[...] = acc[...] + (lsec - vals)

        @plsc.parallel_loop(0, CB, unroll=4)
        def cb_body(cb):
            c0 = cb * 8
            for c in range(8):
                col = jnp.full((L,), c0 + c, jnp.int32)
                stg[ps][cb, c, pl.ds(h * L, L)] = plsc.load_gather(
                    rows[p], [lane, col])

    def quad(q, first, last):
        for k in range(4):
            g = 4 * q + k
            p, ps, h = k % 2, k // 2, k % 2
            wait_gather(p)
            if k in (0, 2) and not first:
                wait_out(ps)
            transpose_loss(g, p, ps, h)
            if k in (1, 3):
                start_out(g - 1, ps)
            if not (last and k >= 2):
                start_gather(g + 2, p)

    start_gather(0, 0)
    start_gather(1, 1)
    quad(0, first=True, last=False)

    def quad_body(q, _):
        quad(q, first=False, last=False)
        return 0

    lax.fori_loop(1, NCHUNK // 4 - 1, quad_body, 0)
    quad(NCHUNK // 4 - 1, first=False, last=True)

    wait_out(0)
    wait_out(1)
    pltpu.sync_copy(acc, part_hbm.at[pl.ds(wid * L, L)])


@functools.partial(jax.jit, static_argnums=())
def kernel(context, targets, table):
    ctx_flat = context.reshape(N_TOK)
    tgt_flat = targets.reshape(N_TOK)
    lse = _row_logsumexp(table)

    mesh = plsc.VectorSubcoreMesh(core_axis_name="c", subcore_axis_name="s")
    sc = pl.kernel(
        _sc_body,
        out_type=(
            jax.ShapeDtypeStruct((CB, RB, 8, 128), jnp.float32),
            jax.ShapeDtypeStruct((NW * L,), jnp.float32),
        ),
        mesh=mesh,
        compiler_params=pltpu.CompilerParams(
            needs_layout_passes=False, use_tc_tiling_on_sc=False),
        scratch_types=[
            pltpu.VMEM((NB,), jnp.int32),
            pltpu.VMEM((NB,), jnp.int32),
            pltpu.VMEM((V,), jnp.float32),
            pltpu.VMEM((CH, V), jnp.float32),
            pltpu.VMEM((CH, V), jnp.float32),
            pltpu.VMEM((CB, 8, 2 * CH), jnp.float32),
            pltpu.VMEM((CB, 8, 2 * CH), jnp.float32),
            pltpu.VMEM((L,), jnp.float32),
            pltpu.SemaphoreType.DMA,
            pltpu.SemaphoreType.DMA,
            pltpu.SemaphoreType.DMA,
            pltpu.SemaphoreType.DMA,
        ],
    )
    out4, partials = sc(table, ctx_flat, tgt_flat, lse)
    logits2 = out4.transpose(1, 3, 0, 2).reshape(N_TOK, V)
    loss = jnp.sum(partials) / jnp.float32(N_TOK)
    return (logits2, loss)
